# baseline (speedup 1.0000x reference)
import jax
import jax.numpy as jnp
from jax.experimental import pallas as pl
from jax.experimental.pallas import tpu as pltpu


def _router_body(x_ref, w1_ref, b1_ref, w2_ref, b2_ref, o_ref, acc):
    j = pl.program_id(1)
    part = jnp.dot(x_ref[...], w1_ref[...], preferred_element_type=jnp.float32)

    @pl.when(j == 0)
    def _store():
        acc[...] = part

    @pl.when(j == 1)
    def _finalize():
        h = jnp.maximum(acc[...] + part + b1_ref[...], 0.0)
        logits = jnp.dot(h, w2_ref[...], preferred_element_type=jnp.float32)
        logits = logits + b2_ref[...]
        m = jnp.max(logits, axis=-1, keepdims=True)
        e = jnp.exp(logits - m)
        o_ref[...] = e / jnp.sum(e, axis=-1, keepdims=True)


def kernel(x, W1, b1, W2, b2):
    M, K = x.shape
    N1 = W1.shape[1]
    N2 = W2.shape[1]
    BM = 2048
    BK = K // 2

    b1r = b1.reshape(1, N1)
    b2r = b2.reshape(1, N2)

    return pl.pallas_call(
        _router_body,
        grid=(M // BM, 2),
        in_specs=[
            pl.BlockSpec((BM, BK), lambda i, j: (i, j)),
            pl.BlockSpec((BK, N1), lambda i, j: (j, 0)),
            pl.BlockSpec((1, N1), lambda i, j: (0, 0)),
            pl.BlockSpec((N1, N2), lambda i, j: (0, 0)),
            pl.BlockSpec((1, N2), lambda i, j: (0, 0)),
        ],
        out_specs=pl.BlockSpec((BM, N2), lambda i, j: (i, 0)),
        out_shape=jax.ShapeDtypeStruct((M, N2), jnp.float32),
        scratch_shapes=[
            pltpu.VMEM((BM, N1), jnp.float32),
        ],
        compiler_params=pltpu.CompilerParams(
            dimension_semantics=("arbitrary", "arbitrary"),
            disable_bounds_checks=True,
        ),
    )(x, W1, b1r, W2, b2r)
